# Initial kernel scaffold; baseline (speedup 1.0000x reference)
import jax
import jax.numpy as jnp
from jax.experimental import pallas as pl


def kernel(x, W_g):
    raise NotImplementedError("write your pallas kernel here")


# speedup vs baseline: 13.4348x; 13.4348x over previous
import functools

import jax
import jax.numpy as jnp
from jax.experimental import pallas as pl

_BT = 1024


def _body(x_ref, w_ref, o_ref):
    y = x_ref[...] - 1.0
    logits = jnp.dot(y, w_ref[...], preferred_element_type=jnp.float32)
    m = jnp.max(logits, axis=-1, keepdims=True)
    denom = jnp.sum(jnp.exp(logits - m), axis=-1, keepdims=True)
    o_ref[...] = y * (1.0 / denom)


@jax.jit
def kernel(x, W_g):
    T, D = x.shape
    E = W_g.shape[1]
    grid = (T // _BT,)
    return pl.pallas_call(
        _body,
        grid=grid,
        in_specs=[
            pl.BlockSpec((_BT, D), lambda i: (i, 0)),
            pl.BlockSpec((D, E), lambda i: (0, 0)),
        ],
        out_specs=pl.BlockSpec((_BT, D), lambda i: (i, 0)),
        out_shape=jax.ShapeDtypeStruct((T, D), x.dtype),
    )(x, W_g)


# speedup vs baseline: 13.7748x; 1.0253x over previous
import functools

import jax
import jax.numpy as jnp
from jax.experimental import pallas as pl

_BT = 2048


def _body(x_ref, w_ref, o_ref):
    y = x_ref[...] - 1.0
    logits = jnp.dot(y, w_ref[...], preferred_element_type=jnp.float32)
    m = jnp.max(logits, axis=-1, keepdims=True)
    denom = jnp.sum(jnp.exp(logits - m), axis=-1, keepdims=True)
    o_ref[...] = y * (1.0 / denom)


@jax.jit
def kernel(x, W_g):
    T, D = x.shape
    E = W_g.shape[1]
    grid = (T // _BT,)
    return pl.pallas_call(
        _body,
        grid=grid,
        in_specs=[
            pl.BlockSpec((_BT, D), lambda i: (i, 0)),
            pl.BlockSpec((D, E), lambda i: (0, 0)),
        ],
        out_specs=pl.BlockSpec((_BT, D), lambda i: (i, 0)),
        out_shape=jax.ShapeDtypeStruct((T, D), x.dtype),
    )(x, W_g)
